# Initial kernel scaffold; baseline (speedup 1.0000x reference)
import jax
import jax.numpy as jnp
from jax.experimental import pallas as pl


def kernel(lat, quant_w, quant_b, codebook, pq_w, pq_b, dec_w1, dec_b1, dec_w2, dec_b2, dec_wout, dec_bout):
    raise NotImplementedError("write your pallas kernel here")


# speedup vs baseline: 1.3269x; 1.3269x over previous
import jax
import jax.numpy as jnp
from jax import lax
from jax.experimental import pallas as pl

B, C, H, W = 8, 64, 32, 32
K = 8192
MB = 1024
KC = 1024
NPIX = B * H * W
NBLK = NPIX // MB
NCH = K // KC


def _gather_body(idx_ref, cbT_ref, pqwT_ref, pqb_ref, h1_ref):
    minidx = idx_ref[...]
    zq = jnp.zeros((MB, C), jnp.float32)
    for k in range(NCH):
        io = lax.broadcasted_iota(jnp.int32, (MB, KC), 1) + k * KC
        oh = (minidx[:, None] == io).astype(jnp.bfloat16)
        cT = cbT_ref[:, k * KC:(k + 1) * KC]
        c_hi = cT.astype(jnp.bfloat16)
        c_lo = (cT - c_hi.astype(jnp.float32)).astype(jnp.bfloat16)
        zq = zq + lax.dot_general(oh, c_hi, (((1,), (1,)), ((), ())),
                                  preferred_element_type=jnp.float32)
        zq = zq + lax.dot_general(oh, c_lo, (((1,), (1,)), ((), ())),
                                  preferred_element_type=jnp.float32)
    h1 = lax.dot_general(zq, pqwT_ref[...], (((1,), (0,)), ((), ())),
                         preferred_element_type=jnp.float32) + pqb_ref[...]
    h1_ref[...] = h1


def _gather_stage(idx, cbT, pqwT, pqb2):
    return pl.pallas_call(
        _gather_body,
        grid=(NBLK,),
        in_specs=[
            pl.BlockSpec((MB,), lambda i: (i,)),
            pl.BlockSpec((C, K), lambda i: (0, 0)),
            pl.BlockSpec((C, C), lambda i: (0, 0)),
            pl.BlockSpec((1, C), lambda i: (0, 0)),
        ],
        out_specs=pl.BlockSpec((MB, C), lambda i: (i, 0)),
        out_shape=jax.ShapeDtypeStruct((NPIX, C), jnp.float32),
    )(idx, cbT, pqwT, pqb2)


def _conv_body(hw, nstripe, cout, relu):
    hp = hw + 2
    rows = hw // nstripe

    def body(x_ref, w_ref, b_ref, o_ref):
        s = pl.program_id(1)
        xst = x_ref[0, pl.ds(s * rows, rows + 3)].reshape((rows + 3) * hp, C)
        acc = jnp.zeros((rows * hp, cout), jnp.float32)
        for dy in range(3):
            for dx in range(3):
                off = dy * hp + dx
                acc = acc + jnp.dot(xst[off:off + rows * hp, :], w_ref[dy, dx],
                                    preferred_element_type=jnp.float32)
        acc = acc + b_ref[...]
        if relu:
            acc = jax.nn.relu(acc)
        o_ref[0] = acc.reshape(rows, hp, cout)[:, :hw]
    return body


def _conv_stage(xpad, wt, b2, hw, nstripe, cout=C, relu=True):
    rows = hw // nstripe
    return pl.pallas_call(
        _conv_body(hw, nstripe, cout, relu),
        grid=(B, nstripe),
        in_specs=[
            pl.BlockSpec((1, hw + 4, hw + 2, C), lambda i, s: (i, 0, 0, 0)),
            pl.BlockSpec((3, 3, C, cout), lambda i, s: (0, 0, 0, 0)),
            pl.BlockSpec((1, cout), lambda i, s: (0, 0)),
        ],
        out_specs=pl.BlockSpec((1, rows, hw, cout), lambda i, s: (i, s, 0, 0)),
        out_shape=jax.ShapeDtypeStruct((B, hw, hw, cout), jnp.float32),
    )(xpad, wt, b2)


def _up2(x):
    return jnp.repeat(jnp.repeat(x, 2, axis=1), 2, axis=2)


def _pad1(x):
    return jnp.pad(x, ((0, 0), (1, 3), (1, 1), (0, 0)))


def kernel(lat, quant_w, quant_b, codebook, pq_w, pq_b,
           dec_w1, dec_b1, dec_w2, dec_b2, dec_wout, dec_bout):
    z = lax.conv_general_dilated(lat, quant_w, window_strides=(1, 1),
                                 padding='SAME',
                                 dimension_numbers=('NCHW', 'OIHW', 'NCHW'))
    z = z + quant_b[None, :, None, None]
    zf = jnp.transpose(z, (0, 2, 3, 1)).reshape(-1, C)
    d = (jnp.sum(zf ** 2, axis=1, keepdims=True)
         + jnp.sum(codebook ** 2, axis=1)[None, :] - 2.0 * zf @ codebook.T)
    idx = jnp.argmin(d, axis=1).astype(jnp.int32)

    pqwT = pq_w.reshape(C, C).T
    h1f = _gather_stage(idx, codebook.T, pqwT, pq_b[None, :])
    h1 = h1f.reshape(B, H, W, C)

    wt1 = jnp.transpose(dec_w1, (2, 3, 1, 0))
    wt2 = jnp.transpose(dec_w2, (2, 3, 1, 0))
    wto = jnp.transpose(dec_wout, (2, 3, 1, 0))
    wto = jnp.pad(wto, ((0, 0), (0, 0), (0, 0), (0, 5)))
    bo = jnp.pad(dec_bout, (0, 5))

    h2 = _conv_stage(_pad1(_up2(h1)), wt1, dec_b1[None, :], 2 * H, 2)
    h3 = _conv_stage(_pad1(_up2(h2)), wt2, dec_b2[None, :], 4 * H, 8)
    out = _conv_stage(_pad1(h3), wto, bo[None, :], 4 * H, 8, cout=8, relu=False)
    return jnp.transpose(out[:, :, :, :3], (0, 3, 1, 2))


# speedup vs baseline: 1.3937x; 1.0503x over previous
import functools
import jax
import jax.numpy as jnp
from jax import lax
from jax.experimental import pallas as pl
from jax.experimental.pallas import tpu as pltpu
from jax.experimental.pallas import tpu_sc as plsc

B, C, H, W = 8, 64, 32, 32
K = 8192
MB = 1024
KC = 1024
NPIX = B * H * W
NBLK = NPIX // MB
NCH = K // KC


def _gather_body(idx_ref, cbT_ref, pqwT_ref, pqb_ref, h1_ref):
    minidx = idx_ref[...]
    zq = jnp.zeros((MB, C), jnp.float32)
    for k in range(NCH):
        io = lax.broadcasted_iota(jnp.int32, (MB, KC), 1) + k * KC
        oh = (minidx[:, None] == io).astype(jnp.bfloat16)
        cT = cbT_ref[:, k * KC:(k + 1) * KC]
        c_hi = cT.astype(jnp.bfloat16)
        c_lo = (cT - c_hi.astype(jnp.float32)).astype(jnp.bfloat16)
        zq = zq + lax.dot_general(oh, c_hi, (((1,), (1,)), ((), ())),
                                  preferred_element_type=jnp.float32)
        zq = zq + lax.dot_general(oh, c_lo, (((1,), (1,)), ((), ())),
                                  preferred_element_type=jnp.float32)
    h1 = lax.dot_general(zq, pqwT_ref[...], (((1,), (0,)), ((), ())),
                         preferred_element_type=jnp.float32) + pqb_ref[...]
    h1_ref[...] = h1


def _gather_stage(idx, cbT, pqwT, pqb2):
    return pl.pallas_call(
        _gather_body,
        grid=(NBLK,),
        in_specs=[
            pl.BlockSpec((MB,), lambda i: (i,)),
            pl.BlockSpec((C, K), lambda i: (0, 0)),
            pl.BlockSpec((C, C), lambda i: (0, 0)),
            pl.BlockSpec((1, C), lambda i: (0, 0)),
        ],
        out_specs=pl.BlockSpec((MB, C), lambda i: (i, 0)),
        out_shape=jax.ShapeDtypeStruct((NPIX, C), jnp.float32),
    )(idx, cbT, pqwT, pqb2)


_SC_NC = 2
_SC_NS = 16
_NW = _SC_NC * _SC_NS
_BPW = NPIX // _NW


def _sc_gather(codebook128, idx):
    mesh = plsc.VectorSubcoreMesh(core_axis_name="c", subcore_axis_name="s")

    @functools.partial(
        pl.kernel, mesh=mesh,
        out_type=jax.ShapeDtypeStruct((NPIX, 128), jnp.float32),
        scratch_types=[
            pltpu.VMEM((_BPW,), jnp.int32),
            pltpu.VMEM((_BPW, 128), jnp.float32),
            pltpu.SemaphoreType.DMA,
        ],
    )
    def k(table_hbm, idx_hbm, out_hbm, idx_v, rows_v, sem):
        wid = lax.axis_index("s") * _SC_NC + lax.axis_index("c")
        base = wid * _BPW
        pltpu.sync_copy(idx_hbm.at[pl.ds(base, _BPW)], idx_v)
        pltpu.async_copy(table_hbm.at[idx_v], rows_v, sem).wait()
        pltpu.sync_copy(rows_v, out_hbm.at[pl.ds(base, _BPW)])

    return k(codebook128, idx)


def _pq_body(zq_ref, pqwT_ref, pqb_ref, h1_ref):
    h1_ref[...] = lax.dot_general(
        zq_ref[...], pqwT_ref[...], (((1,), (0,)), ((), ())),
        preferred_element_type=jnp.float32) + pqb_ref[...]


def _pq_stage(zq, pqwT, pqb2):
    return pl.pallas_call(
        _pq_body,
        grid=(NBLK,),
        in_specs=[
            pl.BlockSpec((MB, C), lambda i: (i, 0)),
            pl.BlockSpec((C, C), lambda i: (0, 0)),
            pl.BlockSpec((1, C), lambda i: (0, 0)),
        ],
        out_specs=pl.BlockSpec((MB, C), lambda i: (i, 0)),
        out_shape=jax.ShapeDtypeStruct((NPIX, C), jnp.float32),
    )(zq, pqwT, pqb2)


def _conv_body(hw, nstripe, cout, relu):
    hp = hw + 2
    rows = hw // nstripe

    def body(x_ref, w_ref, b_ref, o_ref):
        s = pl.program_id(1)
        xst = x_ref[0, pl.ds(s * rows, rows + 3)].reshape((rows + 3) * hp, C)
        acc = jnp.zeros((rows * hp, cout), jnp.float32)
        for dy in range(3):
            for dx in range(3):
                off = dy * hp + dx
                acc = acc + jnp.dot(xst[off:off + rows * hp, :], w_ref[dy, dx],
                                    preferred_element_type=jnp.float32)
        acc = acc + b_ref[...]
        if relu:
            acc = jax.nn.relu(acc)
        o_ref[0] = acc.reshape(rows, hp, cout)[:, :hw]
    return body


def _conv_stage(xpad, wt, b2, hw, nstripe, cout=C, relu=True):
    rows = hw // nstripe
    return pl.pallas_call(
        _conv_body(hw, nstripe, cout, relu),
        grid=(B, nstripe),
        in_specs=[
            pl.BlockSpec((1, hw + 4, hw + 2, C), lambda i, s: (i, 0, 0, 0)),
            pl.BlockSpec((3, 3, C, cout), lambda i, s: (0, 0, 0, 0)),
            pl.BlockSpec((1, cout), lambda i, s: (0, 0)),
        ],
        out_specs=pl.BlockSpec((1, rows, hw, cout), lambda i, s: (i, s, 0, 0)),
        out_shape=jax.ShapeDtypeStruct((B, hw, hw, cout), jnp.float32),
    )(xpad, wt, b2)


def _up2(x):
    return jnp.repeat(jnp.repeat(x, 2, axis=1), 2, axis=2)


def _pad1(x):
    return jnp.pad(x, ((0, 0), (1, 3), (1, 1), (0, 0)))


def kernel(lat, quant_w, quant_b, codebook, pq_w, pq_b,
           dec_w1, dec_b1, dec_w2, dec_b2, dec_wout, dec_bout):
    z = lax.conv_general_dilated(lat, quant_w, window_strides=(1, 1),
                                 padding='SAME',
                                 dimension_numbers=('NCHW', 'OIHW', 'NCHW'))
    z = z + quant_b[None, :, None, None]
    zf = jnp.transpose(z, (0, 2, 3, 1)).reshape(-1, C)
    d = (jnp.sum(zf ** 2, axis=1, keepdims=True)
         + jnp.sum(codebook ** 2, axis=1)[None, :] - 2.0 * zf @ codebook.T)
    idx = jnp.argmin(d, axis=1).astype(jnp.int32)

    cb128 = jnp.pad(codebook, ((0, 0), (0, 128 - C)))
    zq = _sc_gather(cb128, idx)[:, :C]
    pqwT = pq_w.reshape(C, C).T
    h1f = _pq_stage(zq, pqwT, pq_b[None, :])
    h1 = h1f.reshape(B, H, W, C)

    wt1 = jnp.transpose(dec_w1, (2, 3, 1, 0))
    wt2 = jnp.transpose(dec_w2, (2, 3, 1, 0))
    wto = jnp.transpose(dec_wout, (2, 3, 1, 0))
    wto = jnp.pad(wto, ((0, 0), (0, 0), (0, 0), (0, 5)))
    bo = jnp.pad(dec_bout, (0, 5))

    h2 = _conv_stage(_pad1(_up2(h1)), wt1, dec_b1[None, :], 2 * H, 2)
    h3 = _conv_stage(_pad1(_up2(h2)), wt2, dec_b2[None, :], 4 * H, 8)
    out = _conv_stage(_pad1(h3), wto, bo[None, :], 4 * H, 8, cout=8, relu=False)
    return jnp.transpose(out[:, :, :, :3], (0, 3, 1, 2))
